# baseline (speedup 1.0000x reference)
import math

import jax
import jax.numpy as jnp
from jax import lax
from jax.experimental import pallas as pl
from jax.experimental.pallas import tpu as pltpu

H = 4
SPARSITY_RATIO = 0.1
MEMORY_PRESSURE = 0.5
THRESH = SPARSITY_RATIO * (1.0 + MEMORY_PRESSURE)
NEG = -1e30
BQ = 512

_DNT = (((1,), (1,)), ((), ()))


def _mmt(x, w):
    return lax.dot_general(x, w, _DNT, preferred_element_type=jnp.float32)


def _bf(x):
    return x.astype(jnp.bfloat16)


def _proj_kernel(k_ref, v_ref, q_ref, wk_ref, wv_ref, ipb_ref,
                 w1_ref, b1_ref, w2_ref, b2_ref,
                 kp_ref, vp_ref, impt_ref):
    kp_ref[...] = _bf(_mmt(k_ref[...], wk_ref[...]) + ipb_ref[1])
    vp_ref[...] = _bf(_mmt(v_ref[...], wv_ref[...]) + ipb_ref[2])
    hid = jnp.maximum(_mmt(q_ref[...], w1_ref[...]) + b1_ref[...], 0.0)
    logit = _mmt(w2_ref[...], hid) + b2_ref[...]
    impt_ref[...] = jax.nn.sigmoid(logit)


def _attn_kernel(q_ref, impt_ref, kp_ref, vp_ref,
                 wq_ref, ipb_ref, ow_ref, ob_ref, out_ref):
    j = pl.program_id(0)
    d = q_ref.shape[1]
    hd = d // H

    imp_row = impt_ref[...]
    validk = (imp_row > THRESH).astype(jnp.float32)
    count = jnp.sum(validk)
    use_fb = count == 0.0
    fb_row = (lax.broadcasted_iota(jnp.int32, imp_row.shape, 1) < 32
              ).astype(jnp.float32)
    validk = jnp.where(use_fb, fb_row, validk)
    kbias = (validk - 1.0) * (-NEG)

    impq = jnp.reshape(impt_ref[0, pl.ds(j * BQ, BQ)], (BQ, 1))
    rows = lax.broadcasted_iota(jnp.int32, (BQ, 1), 0) + j * BQ
    validq = jnp.where(use_fb, (rows < 32).astype(jnp.float32),
                       (impq > THRESH).astype(jnp.float32))

    scale = 1.0 / math.sqrt(hd)
    qp = _bf((_mmt(q_ref[...], wq_ref[...]) + ipb_ref[0]) * scale)
    acc = jnp.zeros(out_ref.shape, jnp.float32)
    for h in range(H):
        sl = slice(h * hd, (h + 1) * hd)
        s = lax.dot_general(qp[:, sl], kp_ref[:, sl], _DNT,
                            preferred_element_type=jnp.float32)
        s = s + kbias
        m = jnp.max(s, axis=1, keepdims=True)
        p = jnp.exp(s - m)
        l = jnp.sum(p, axis=1, keepdims=True)
        ctx = lax.dot_general(_bf(p), vp_ref[:, sl],
                              (((1,), (0,)), ((), ())),
                              preferred_element_type=jnp.float32) / l
        acc = acc + _mmt(_bf(ctx), ow_ref[:, sl])
    out_ref[...] = (acc + ob_ref[...]) * validq


def kernel(q, k, v, W1, b1, W2, b2, in_proj_w, in_proj_b, out_w, out_b):
    batch, seq, d = q.shape
    n = batch * seq
    nblk = n // BQ
    dh = W1.shape[0]

    q2 = q.reshape(n, d)
    k2 = k.reshape(n, d)
    v2 = v.reshape(n, d)
    ipb = in_proj_b.reshape(3, 1, d)

    full = lambda shape: pl.BlockSpec(shape, lambda i: (0,) * len(shape))
    blk = pl.BlockSpec((BQ, d), lambda i: (i, 0))
    ipw_at = lambda j: pl.BlockSpec((d, d), lambda i: (j, 0))
    par = pltpu.CompilerParams(dimension_semantics=("parallel",))

    kp, vp, impt = pl.pallas_call(
        _proj_kernel,
        grid=(nblk,),
        in_specs=[blk, blk, blk, ipw_at(1), ipw_at(2), full((3, 1, d)),
                  full((dh, d)), full((1, dh)), full((1, dh)), full((1, 1))],
        out_specs=[blk, blk, pl.BlockSpec((1, BQ), lambda i: (0, i))],
        out_shape=[jax.ShapeDtypeStruct((n, d), jnp.bfloat16),
                   jax.ShapeDtypeStruct((n, d), jnp.bfloat16),
                   jax.ShapeDtypeStruct((1, n), jnp.float32)],
        compiler_params=par,
    )(k2, v2, q2, in_proj_w, in_proj_w, ipb,
      W1, b1.reshape(1, dh), W2, b2.reshape(1, 1))

    out = pl.pallas_call(
        _attn_kernel,
        grid=(nblk,),
        in_specs=[blk, full((1, n)), full((n, d)), full((n, d)),
                  ipw_at(0), full((3, 1, d)), full((d, d)), full((1, d))],
        out_specs=blk,
        out_shape=jax.ShapeDtypeStruct((n, d), jnp.float32),
        compiler_params=par,
    )(q2, impt, kp, vp, in_proj_w, ipb, out_w, out_b.reshape(1, d))

    return out.reshape(batch, seq, d)


# speedup vs baseline: 1.0550x; 1.0550x over previous
import math

import jax
import jax.numpy as jnp
from jax import lax
from jax.experimental import pallas as pl
from jax.experimental.pallas import tpu as pltpu

H = 4
SPARSITY_RATIO = 0.1
MEMORY_PRESSURE = 0.5
THRESH = SPARSITY_RATIO * (1.0 + MEMORY_PRESSURE)
NEG = -1e30
BQ = 512

_DNT = (((1,), (1,)), ((), ()))


def _mmt(x, w):
    return lax.dot_general(x, w, _DNT, preferred_element_type=jnp.float32)


def _bf(x):
    return x.astype(jnp.bfloat16)


def _fused_kernel(k_ref, v_ref, q_ref, wk_ref, wv_ref, wq_ref,
                  w1_ref, b1_ref, w2_ref, b2_ref, ipb_ref, ow_ref, ob_ref,
                  out_ref, kp_s, vp_s, qp_s, imp_s, wk_b, wv_b, wq_b, ow_b):
    i = pl.program_id(0)
    nblk = pl.num_programs(0) // 2
    d = k_ref.shape[1]
    hd = d // H

    @pl.when(i == 0)
    def _pack_weights():
        wk_b[...] = _bf(wk_ref[...])
        wv_b[...] = _bf(wv_ref[...])
        wq_b[...] = _bf(wq_ref[...])
        ow_b[...] = _bf(ow_ref[...])

    @pl.when(i < nblk)
    def _proj_phase():
        rows = pl.ds(i * BQ, BQ)
        scale = 1.0 / math.sqrt(hd)
        kp_s[rows, :] = _bf(_mmt(_bf(k_ref[...]), wk_b[...]) + ipb_ref[1])
        vp_s[rows, :] = _bf(_mmt(_bf(v_ref[...]), wv_b[...]) + ipb_ref[2])
        qb = q_ref[...]
        qp_s[rows, :] = _bf((_mmt(_bf(qb), wq_b[...]) + ipb_ref[0]) * scale)
        hid = jnp.maximum(_mmt(qb, w1_ref[...]) + b1_ref[...], 0.0)
        logit = _mmt(w2_ref[...], hid) + b2_ref[...]
        imp_s[:, pl.ds(i * BQ, BQ)] = jax.nn.sigmoid(logit)

    @pl.when(i >= nblk)
    def _attn_phase():
        j = i - nblk

        imp_row = imp_s[...]
        validk = (imp_row > THRESH).astype(jnp.float32)
        count = jnp.sum(validk)
        use_fb = count == 0.0
        fb_row = (lax.broadcasted_iota(jnp.int32, imp_row.shape, 1) < 32
                  ).astype(jnp.float32)
        validk = jnp.where(use_fb, fb_row, validk)
        kbias = (validk - 1.0) * (-NEG)

        impq = jnp.reshape(imp_s[0, pl.ds(j * BQ, BQ)], (BQ, 1))
        rows = lax.broadcasted_iota(jnp.int32, (BQ, 1), 0) + j * BQ
        validq = jnp.where(use_fb, (rows < 32).astype(jnp.float32),
                           (impq > THRESH).astype(jnp.float32))

        qrows = pl.ds(j * BQ, BQ)
        acc = jnp.zeros(out_ref.shape, jnp.float32)
        for h in range(H):
            sl = slice(h * hd, (h + 1) * hd)
            s = lax.dot_general(qp_s[qrows, sl], kp_s[:, sl], _DNT,
                                preferred_element_type=jnp.float32)
            s = s + kbias
            m = jnp.max(s, axis=1, keepdims=True)
            p = jnp.exp(s - m)
            l = jnp.sum(p, axis=1, keepdims=True)
            ctx = lax.dot_general(_bf(p), vp_s[:, sl],
                                  (((1,), (0,)), ((), ())),
                                  preferred_element_type=jnp.float32) / l
            acc = acc + _mmt(_bf(ctx), ow_b[:, sl])
        out_ref[...] = (acc + ob_ref[...]) * validq


def kernel(q, k, v, W1, b1, W2, b2, in_proj_w, in_proj_b, out_w, out_b):
    batch, seq, d = q.shape
    n = batch * seq
    nblk = n // BQ
    dh = W1.shape[0]

    q2 = q.reshape(n, d)
    k2 = k.reshape(n, d)
    v2 = v.reshape(n, d)
    ipb = in_proj_b.reshape(3, 1, d)

    full = lambda shape: pl.BlockSpec(shape, lambda i: (0,) * len(shape))
    clamp = pl.BlockSpec((BQ, d), lambda i: (jnp.minimum(i, nblk - 1), 0))
    both = pl.BlockSpec((BQ, d),
                        lambda i: (jnp.where(i < nblk, i, i - nblk), 0))
    outsp = pl.BlockSpec((BQ, d),
                         lambda i: (jnp.maximum(i - nblk, 0), 0))
    ipw_at = lambda j: pl.BlockSpec((d, d), lambda i: (j, 0))

    out = pl.pallas_call(
        _fused_kernel,
        grid=(2 * nblk,),
        in_specs=[clamp, clamp, clamp,
                  ipw_at(1), ipw_at(2), ipw_at(0),
                  full((dh, d)), full((1, dh)), full((1, dh)), full((1, 1)),
                  full((3, 1, d)), full((d, d)), full((1, d))],
        out_specs=outsp,
        out_shape=jax.ShapeDtypeStruct((n, d), jnp.float32),
        compiler_params=pltpu.CompilerParams(
            vmem_limit_bytes=100 * 1024 * 1024),
        scratch_shapes=[pltpu.VMEM((n, d), jnp.bfloat16),
                        pltpu.VMEM((n, d), jnp.bfloat16),
                        pltpu.VMEM((n, d), jnp.bfloat16),
                        pltpu.VMEM((1, n), jnp.float32),
                        pltpu.VMEM((d, d), jnp.bfloat16),
                        pltpu.VMEM((d, d), jnp.bfloat16),
                        pltpu.VMEM((d, d), jnp.bfloat16),
                        pltpu.VMEM((d, d), jnp.bfloat16)],
    )(k2, v2, q2, in_proj_w, in_proj_w, in_proj_w,
      W1, b1.reshape(1, dh), W2, b2.reshape(1, 1), ipb,
      out_w, out_b.reshape(1, d))

    return out.reshape(batch, seq, d)


# speedup vs baseline: 1.1628x; 1.1022x over previous
import math

import jax
import jax.numpy as jnp
from jax import lax
from jax.experimental import pallas as pl
from jax.experimental.pallas import tpu as pltpu

H = 4
SPARSITY_RATIO = 0.1
MEMORY_PRESSURE = 0.5
THRESH = SPARSITY_RATIO * (1.0 + MEMORY_PRESSURE)
NEG = -1e30
BQ = 512

_DNT = (((1,), (1,)), ((), ()))


def _mmt(x, w):
    return lax.dot_general(x, w, _DNT, preferred_element_type=jnp.float32)


def _bf(x):
    return x.astype(jnp.bfloat16)


def _fused_kernel(k_ref, v_ref, q_ref, wk_ref, wv_ref, wq_ref,
                  w1_ref, b1_ref, w2_ref, b2_ref, ipb_ref, ow_ref, ob_ref,
                  out_ref, kp_s, vp_s, imp_s, wk_b, wv_b, wq_b, ow_b):
    i = pl.program_id(0)
    nblk = pl.num_programs(0) // 2
    d = k_ref.shape[1]
    hd = d // H

    @pl.when(i == 0)
    def _pack_weights():
        wk_b[...] = _bf(wk_ref[...])
        wv_b[...] = _bf(wv_ref[...])
        wq_b[...] = _bf(wq_ref[...])
        ow_b[...] = _bf(ow_ref[...])

    @pl.when(i < nblk)
    def _proj_phase():
        rows = pl.ds(i * BQ, BQ)
        kp_s[rows, :] = _bf(_mmt(_bf(k_ref[...]), wk_b[...]) + ipb_ref[1])
        vp_s[rows, :] = _bf(_mmt(_bf(v_ref[...]), wv_b[...]) + ipb_ref[2])
        hid = jnp.maximum(_mmt(q_ref[...], w1_ref[...]) + b1_ref[...], 0.0)
        logit = _mmt(w2_ref[...], hid) + b2_ref[...]
        imp_s[:, pl.ds(i * BQ, BQ)] = jax.nn.sigmoid(logit)

    @pl.when(i >= nblk)
    def _attn_phase():
        j = i - nblk
        n_tok = kp_s.shape[0]

        imp_row = imp_s[...]
        validk = (imp_row > THRESH).astype(jnp.float32)
        count = jnp.sum(validk)
        all_valid = count == jnp.float32(n_tok)

        scale = 1.0 / math.sqrt(hd)
        qp = _bf((_mmt(_bf(q_ref[...]), wq_b[...]) + ipb_ref[0]) * scale)

        def _score(h):
            sl = slice(h * hd, (h + 1) * hd)
            return lax.dot_general(qp[:, sl], kp_s[:, sl], _DNT,
                                   preferred_element_type=jnp.float32)

        def _tail(h, p, l):
            sl = slice(h * hd, (h + 1) * hd)
            ctx = lax.dot_general(_bf(p), vp_s[:, sl],
                                  (((1,), (0,)), ((), ())),
                                  preferred_element_type=jnp.float32) / l
            return _mmt(_bf(ctx), ow_b[:, sl])

        @pl.when(all_valid)
        def _dense_path():
            acc = jnp.zeros(out_ref.shape, jnp.float32)
            s_cur = _score(0)
            for h in range(H):
                s_next = _score(h + 1) if h + 1 < H else None
                m = jnp.max(s_cur, axis=1, keepdims=True)
                p = jnp.exp(s_cur - m)
                l = jnp.sum(p, axis=1, keepdims=True)
                acc = acc + _tail(h, p, l)
                s_cur = s_next
            out_ref[...] = acc + ob_ref[...]

        @pl.when(jnp.logical_not(all_valid))
        def _masked_path():
            use_fb = count == 0.0
            fb_row = (lax.broadcasted_iota(jnp.int32, imp_row.shape, 1) < 32
                      ).astype(jnp.float32)
            vk = jnp.where(use_fb, fb_row, validk)
            kbias = (vk - 1.0) * (-NEG)

            impq = jnp.reshape(imp_s[0, pl.ds(j * BQ, BQ)], (BQ, 1))
            rows = lax.broadcasted_iota(jnp.int32, (BQ, 1), 0) + j * BQ
            validq = jnp.where(use_fb, (rows < 32).astype(jnp.float32),
                               (impq > THRESH).astype(jnp.float32))

            acc = jnp.zeros(out_ref.shape, jnp.float32)
            for h in range(H):
                s = _score(h) + kbias
                m = jnp.max(s, axis=1, keepdims=True)
                p = jnp.exp(s - m)
                l = jnp.sum(p, axis=1, keepdims=True)
                acc = acc + _tail(h, p, l)
            out_ref[...] = (acc + ob_ref[...]) * validq


def kernel(q, k, v, W1, b1, W2, b2, in_proj_w, in_proj_b, out_w, out_b):
    batch, seq, d = q.shape
    n = batch * seq
    nblk = n // BQ
    dh = W1.shape[0]

    q2 = q.reshape(n, d)
    k2 = k.reshape(n, d)
    v2 = v.reshape(n, d)
    ipb = in_proj_b.reshape(3, 1, d)

    full = lambda shape: pl.BlockSpec(shape, lambda i: (0,) * len(shape))
    clamp = pl.BlockSpec((BQ, d), lambda i: (jnp.minimum(i, nblk - 1), 0))
    both = pl.BlockSpec((BQ, d),
                        lambda i: (jnp.where(i < nblk, i, i - nblk), 0))
    outsp = pl.BlockSpec((BQ, d),
                         lambda i: (jnp.maximum(i - nblk, 0), 0))
    ipw_at = lambda j: pl.BlockSpec((d, d), lambda i: (j, 0))

    out = pl.pallas_call(
        _fused_kernel,
        grid=(2 * nblk,),
        in_specs=[clamp, clamp, both,
                  ipw_at(1), ipw_at(2), ipw_at(0),
                  full((dh, d)), full((1, dh)), full((1, dh)), full((1, 1)),
                  full((3, 1, d)), full((d, d)), full((1, d))],
        out_specs=outsp,
        out_shape=jax.ShapeDtypeStruct((n, d), jnp.float32),
        compiler_params=pltpu.CompilerParams(
            vmem_limit_bytes=100 * 1024 * 1024),
        scratch_shapes=[pltpu.VMEM((n, d), jnp.bfloat16),
                        pltpu.VMEM((n, d), jnp.bfloat16),
                        pltpu.VMEM((1, n), jnp.float32),
                        pltpu.VMEM((d, d), jnp.bfloat16),
                        pltpu.VMEM((d, d), jnp.bfloat16),
                        pltpu.VMEM((d, d), jnp.bfloat16),
                        pltpu.VMEM((d, d), jnp.bfloat16)],
    )(k2, v2, q2, in_proj_w, in_proj_w, in_proj_w,
      W1, b1.reshape(1, dh), W2, b2.reshape(1, 1), ipb,
      out_w, out_b.reshape(1, d))

    return out.reshape(batch, seq, d)
